# baseline (speedup 1.0000x reference)
import jax
import jax.numpy as jnp
from jax import lax
from jax.experimental import pallas as pl
from jax.experimental.pallas import tpu as pltpu
from jax.experimental.pallas import tpu_sc as plsc

VOCAB = 1000000
DIM = 64
B = 4096
X_MAX = 100.0
ALPHA = 0.75

NC = 2
NS = 16
NW = NC * NS
BPW = B // NW

_MESH = plsc.VectorSubcoreMesh(core_axis_name="c", subcore_axis_name="s")


def _sc_gather_body(widx_hbm, cidx_hbm, wtab_hbm, ctab_hbm,
                    wbias_hbm, cbias_hbm,
                    wrows_out, crows_out, wb_out, cb_out,
                    widx_v, cidx_v, wrows_v, crows_v, wb_v, cb_v,
                    sem1, sem2, sem3, sem4):
    wid = lax.axis_index("s") * NC + lax.axis_index("c")
    base = wid * BPW
    pltpu.sync_copy(widx_hbm.at[pl.ds(base, BPW)], widx_v)
    pltpu.sync_copy(cidx_hbm.at[pl.ds(base, BPW)], cidx_v)
    c1 = pltpu.async_copy(wtab_hbm.at[widx_v], wrows_v, sem1)
    c2 = pltpu.async_copy(ctab_hbm.at[cidx_v], crows_v, sem2)
    c3 = pltpu.async_copy(wbias_hbm.at[widx_v], wb_v, sem3)
    c4 = pltpu.async_copy(cbias_hbm.at[cidx_v], cb_v, sem4)
    c1.wait()
    c2.wait()
    c3.wait()
    c4.wait()
    pltpu.sync_copy(wrows_v, wrows_out.at[pl.ds(base, BPW)])
    pltpu.sync_copy(crows_v, crows_out.at[pl.ds(base, BPW)])
    pltpu.sync_copy(wb_v, wb_out.at[pl.ds(base, BPW)])
    pltpu.sync_copy(cb_v, cb_out.at[pl.ds(base, BPW)])


_sc_gather = pl.kernel(
    _sc_gather_body,
    out_type=[
        jax.ShapeDtypeStruct((B, DIM), jnp.float32),
        jax.ShapeDtypeStruct((B, DIM), jnp.float32),
        jax.ShapeDtypeStruct((B,), jnp.float32),
        jax.ShapeDtypeStruct((B,), jnp.float32),
    ],
    mesh=_MESH,
    scratch_types=[
        pltpu.VMEM((BPW,), jnp.int32),
        pltpu.VMEM((BPW,), jnp.int32),
        pltpu.VMEM((BPW, DIM), jnp.float32),
        pltpu.VMEM((BPW, DIM), jnp.float32),
        pltpu.VMEM((BPW,), jnp.float32),
        pltpu.VMEM((BPW,), jnp.float32),
        pltpu.SemaphoreType.DMA,
        pltpu.SemaphoreType.DMA,
        pltpu.SemaphoreType.DMA,
        pltpu.SemaphoreType.DMA,
    ],
    compiler_params=pltpu.CompilerParams(use_tc_tiling_on_sc=False),
)

_R = 32


def _tc_loss_body(wrows_ref, crows_ref, wb_ref, cb_ref, cooc_ref, out_ref):
    dots = jnp.sum(wrows_ref[:, :] * crows_ref[:, :], axis=1)
    dots = dots.reshape(_R, B // _R)
    b = wb_ref[:, :] + cb_ref[:, :]
    cc = cooc_ref[:, :]
    e = jnp.minimum(jnp.power(cc * (1.0 / X_MAX), ALPHA), 1.0)
    a = dots - jnp.log(cc + 1.0)
    s1 = jnp.sum(e * a * a)
    s2 = jnp.sum(e * a)
    s3 = jnp.sum(b)
    s4 = jnp.sum(b * b)
    s5 = jnp.sum(e)
    loss = s1 / B + (2.0 * s2 * s3 + s5 * s4) / (B * B)
    out_ref[:, :] = jnp.reshape(loss, (1, 1))


_tc_loss = pl.pallas_call(
    _tc_loss_body,
    out_shape=jax.ShapeDtypeStruct((1, 1), jnp.float32),
)


def kernel(word_input, context_input, coocurrence_count, word_emb_table,
           word_bias_table, context_emb_table, context_bias_table):
    wrows, crows, wb, cb = _sc_gather(
        word_input, context_input, word_emb_table, context_emb_table,
        word_bias_table[:, 0], context_bias_table[:, 0])
    loss = _tc_loss(wrows, crows, wb.reshape(_R, B // _R),
                    cb.reshape(_R, B // _R),
                    coocurrence_count.reshape(_R, B // _R))
    return loss.reshape(())
